# baseline (speedup 1.0000x reference)
import math

import jax
import jax.numpy as jnp
import numpy as np
from jax.experimental import pallas as pl
from jax.experimental.pallas import tpu as pltpu

_PI = np.float32(math.pi)
_L = 10


_NB = 16

_ATAN_C = (1.0, -0.33333206, 0.19995107, -0.14222199, 0.10710357,
           -0.07647073, 0.04433092, -0.01705405, 0.00309145)


def _atan2_pos(st, ct):
    act = jnp.abs(ct)
    lo = jnp.minimum(st, act)
    hi = jnp.maximum(st, act)
    q = lo / hi
    z = q * q
    acc = jnp.full_like(z, _ATAN_C[-1])
    for c in _ATAN_C[-2::-1]:
        acc = acc * z + c
    p = acc * q
    alpha = jnp.where(act >= st, p, 0.5 * _PI - p)
    return jnp.where(ct >= 0.0, alpha, _PI - alpha)


def _body(x_ref, o_ref):
    x0 = x_ref[:, 0]
    x1 = x_ref[:, 1]
    x2 = x_ref[:, 2]

    st = jnp.sqrt(x0 * x0 + x1 * x1) + 1e-5
    ct = x2 + 1e-5
    theta = _atan2_pos(st, ct)

    b1 = theta
    t2 = 2.0 * theta
    b2 = jnp.where(t2 > _PI, t2 - 2.0 * _PI, t2)
    t3 = 3.0 * theta
    b3 = jnp.where(t3 > _PI, t3 - 2.0 * _PI, t3)

    sg2 = jnp.where(ct >= 0.0, _PI, -_PI)
    sg3 = jnp.where(3.0 * ct * ct >= st * st, _PI, -_PI)

    xa = x0 + 1e-5
    ya = x1 + 1e-5
    a2 = xa * xa
    y2 = ya * ya

    zero = jnp.zeros_like(xa)
    neg_pi = jnp.full_like(xa, -_PI)

    phases = [
        None,
        jnp.where(xa < 0.0, neg_pi, zero),
        b1 - _PI,
        jnp.where(2.0 * a2 < y2, neg_pi, zero),
        jnp.where(xa > 0.0, b1 - _PI, b1),
        b2,
        jnp.where(xa > 0.0,
                  jnp.where(2.0 * a2 < 3.0 * y2, neg_pi, zero),
                  jnp.where(2.0 * a2 > 3.0 * y2, neg_pi, zero)),
        jnp.where(4.0 * a2 > y2, b1 - _PI, b1),
        jnp.where(xa < 0.0, b2 - sg2, b2),
        b3 - sg3,
    ]

    o_ref[:, 0] = jnp.zeros_like(o_ref[:, 0])
    for i in range(_NB):
        for ch in range(1, _L):
            ph = phases[ch][i]
            o_ref[i, ch] = jax.lax.dot_general(
                ph, ph, (((1,), (1,)), ((), ())),
                preferred_element_type=jnp.float32)


def kernel(x):
    N, C, T, V = x.shape
    xt = jnp.swapaxes(x, 2, 3)
    return pl.pallas_call(
        _body,
        grid=(N // _NB,),
        in_specs=[pl.BlockSpec((_NB, C, V, T), lambda n: (n, 0, 0, 0))],
        out_specs=pl.BlockSpec((_NB, _L, V, V), lambda n: (n, 0, 0, 0)),
        out_shape=jax.ShapeDtypeStruct((N, _L, V, V), x.dtype),
        compiler_params=pltpu.CompilerParams(
            dimension_semantics=("parallel",)),
    )(xt)


# speedup vs baseline: 1.0021x; 1.0021x over previous
import math

import jax
import jax.numpy as jnp
import numpy as np
from jax.experimental import pallas as pl
from jax.experimental.pallas import tpu as pltpu

_PI = np.float32(math.pi)
_L = 10


_NB = 8

_ATAN_C = (1.0, -0.33333206, 0.19995107, -0.14222199, 0.10710357,
           -0.07647073, 0.04433092, -0.01705405, 0.00309145)


def _atan2_pos(st, ct):
    act = jnp.abs(ct)
    lo = jnp.minimum(st, act)
    hi = jnp.maximum(st, act)
    q = lo / hi
    z = q * q
    acc = jnp.full_like(z, _ATAN_C[-1])
    for c in _ATAN_C[-2::-1]:
        acc = acc * z + c
    p = acc * q
    alpha = jnp.where(act >= st, p, 0.5 * _PI - p)
    return jnp.where(ct >= 0.0, alpha, _PI - alpha)


def _body(x_ref, o_ref):
    x0 = x_ref[:, 0]
    x1 = x_ref[:, 1]
    x2 = x_ref[:, 2]

    st = jnp.sqrt(x0 * x0 + x1 * x1) + 1e-5
    ct = x2 + 1e-5
    theta = _atan2_pos(st, ct)

    b1 = theta
    t2 = 2.0 * theta
    b2 = jnp.where(t2 > _PI, t2 - 2.0 * _PI, t2)
    t3 = 3.0 * theta
    b3 = jnp.where(t3 > _PI, t3 - 2.0 * _PI, t3)

    sg2 = jnp.where(ct >= 0.0, _PI, -_PI)
    sg3 = jnp.where(3.0 * ct * ct >= st * st, _PI, -_PI)

    xa = x0 + 1e-5
    ya = x1 + 1e-5
    a2 = xa * xa
    y2 = ya * ya

    zero = jnp.zeros_like(xa)
    neg_pi = jnp.full_like(xa, -_PI)

    phases = [
        None,
        jnp.where(xa < 0.0, neg_pi, zero),
        b1 - _PI,
        jnp.where(2.0 * a2 < y2, neg_pi, zero),
        jnp.where(xa > 0.0, b1 - _PI, b1),
        b2,
        jnp.where(xa > 0.0,
                  jnp.where(2.0 * a2 < 3.0 * y2, neg_pi, zero),
                  jnp.where(2.0 * a2 > 3.0 * y2, neg_pi, zero)),
        jnp.where(4.0 * a2 > y2, b1 - _PI, b1),
        jnp.where(xa < 0.0, b2 - sg2, b2),
        b3 - sg3,
    ]

    o_ref[:, 0] = jnp.zeros_like(o_ref[:, 0])
    for i in range(_NB):
        for ch in range(1, _L):
            ph = phases[ch][i]
            o_ref[i, ch] = jax.lax.dot_general(
                ph, ph, (((1,), (1,)), ((), ())),
                preferred_element_type=jnp.float32)


def kernel(x):
    N, C, T, V = x.shape
    xt = jnp.swapaxes(x, 2, 3)
    return pl.pallas_call(
        _body,
        grid=(N // _NB,),
        in_specs=[pl.BlockSpec((_NB, C, V, T), lambda n: (n, 0, 0, 0))],
        out_specs=pl.BlockSpec((_NB, _L, V, V), lambda n: (n, 0, 0, 0)),
        out_shape=jax.ShapeDtypeStruct((N, _L, V, V), x.dtype),
        compiler_params=pltpu.CompilerParams(
            dimension_semantics=("parallel",)),
    )(xt)
